# baseline (speedup 1.0000x reference)
import functools

import jax
import jax.numpy as jnp
from jax import lax
from jax.experimental import pallas as pl
from jax.experimental.pallas import tpu as pltpu
from jax.experimental.pallas import tpu_sc as plsc

K = 100000
Q = 16
F = 64
BK = 5000
NB = K // BK


def _argmin_body(x_ref, audio_ref, dummy_ref, idx_ref, loss_ref, rmin_ref, ridx_ref):
    pid = pl.program_id(0)
    xm = jnp.mean(x_ref[...], axis=1)
    blk = audio_ref[...]
    dots = lax.dot_general(blk, xm, (((1,), (1,)), ((), ())),
                           preferred_element_type=jnp.float32,
                           precision=lax.Precision.HIGHEST)
    rnorm = jnp.sum(blk * blk, axis=1, keepdims=True)
    scores = rnorm - 2.0 * dots

    bmin = jnp.min(scores, axis=0, keepdims=True)
    rows = lax.broadcasted_iota(jnp.int32, (BK, Q), 0) + pid * BK
    bidx = jnp.min(jnp.where(scores == bmin, rows, K), axis=0, keepdims=True)

    @pl.when(pid == 0)
    def _():
        rmin_ref[...] = bmin
        ridx_ref[...] = bidx

    @pl.when(pid > 0)
    def _():
        upd = bmin < rmin_ref[...]
        rmin_ref[...] = jnp.where(upd, bmin, rmin_ref[...])
        ridx_ref[...] = jnp.where(upd, bidx, ridx_ref[...])

    @pl.when(pid == NB - 1)
    def _():
        idx_ref[...] = ridx_ref[...]
        loss_ref[...] = jnp.sum(dummy_ref[...], keepdims=True)


_argmin_call = pl.pallas_call(
    _argmin_body,
    grid=(NB,),
    in_specs=[
        pl.BlockSpec((Q, 20, F), lambda i: (0, 0, 0)),
        pl.BlockSpec((BK, F), lambda i: (i, 0)),
        pl.BlockSpec((1, 1), lambda i: (0, 0)),
    ],
    out_specs=[
        pl.BlockSpec((1, Q), lambda i: (0, 0)),
        pl.BlockSpec((1, 1), lambda i: (0, 0)),
    ],
    out_shape=[
        jax.ShapeDtypeStruct((1, Q), jnp.int32),
        jax.ShapeDtypeStruct((1, 1), jnp.float32),
    ],
    scratch_shapes=[
        pltpu.VMEM((1, Q), jnp.float32),
        pltpu.VMEM((1, Q), jnp.int32),
    ],
)


@functools.lru_cache(maxsize=1)
def _make_gather():
    mesh = plsc.VectorSubcoreMesh(core_axis_name="c", subcore_axis_name="s")
    rows_per_core = Q // 2
    D = 20 * 32

    @functools.partial(
        pl.kernel, mesh=mesh,
        out_type=jax.ShapeDtypeStruct((Q, D), jnp.float32),
        scratch_types=[
            pltpu.VMEM((rows_per_core,), jnp.int32),
            pltpu.VMEM((rows_per_core, D), jnp.float32),
            pltpu.SemaphoreType.DMA,
        ],
    )
    def gather_k(pose_hbm, idx_hbm, out_hbm, idx_v, rows_v, sem):
        cid = lax.axis_index("c")
        sid = lax.axis_index("s")

        @pl.when(sid == 0)
        def _():
            base = cid * rows_per_core
            pltpu.sync_copy(idx_hbm.at[pl.ds(base, rows_per_core)], idx_v)
            pltpu.async_copy(pose_hbm.at[idx_v], rows_v, sem).wait()
            pltpu.sync_copy(rows_v, out_hbm.at[pl.ds(base, rows_per_core)])

    return gather_k


@jax.jit
def kernel(x, y, audio, pose, dummy):
    idx2d, loss = _argmin_call(x[0], audio, dummy.reshape(1, 1))
    pose2d = pose.reshape(K, 20 * 32)
    out = _make_gather()(pose2d, idx2d.reshape(Q)).reshape(Q, 20, 32)
    return (out, loss[0, 0])
